# Initial kernel scaffold; baseline (speedup 1.0000x reference)
import jax
import jax.numpy as jnp
from jax.experimental import pallas as pl


def kernel(x, visual_params, semantic_params, episodic_params, ws_params):
    raise NotImplementedError("write your pallas kernel here")


# speedup vs baseline: 1.8720x; 1.8720x over previous
import jax
import jax.numpy as jnp
from jax.experimental import pallas as pl
from jax.experimental.pallas import tpu as pltpu

B, S, INPUT_DIM, WS = 64, 256, 2, 64
DM = INPUT_DIM + WS
DI = 2 * DM
DCONV = 4
DT_RANK = 5
TEMP = 0.05
LN_EPS = 1e-5
DS = (16, 64, 128)
NCORE = 2
BB = B // NCORE


def _thalamus_kernel(
    x_ref,
    inw_v, convw_v, convb_v, xw_v, dtb_v, A_v, D_v, outw_v,
    inw_s, convw_s, convb_s, xw_s, dtb_s, A_s, D_s, outw_s,
    inw_e, convw_e, convb_e, xw_e, dtb_e, A_e, D_e, outw_e,
    projw_ref, projb_ref, gatew_ref, gateb_ref,
    lng_ref, lnb_ref, headw_ref, headb_ref,
    outs_ref,
    gates_ref,
    conv_v, conv_s, conv_e,
    ssm_v, ssm_s, ssm_e,
):
    def mamba_step(combined, inw, convw, convb, xw, dtb, A, D, outw,
                   conv_ref, ssm_ref, ds, first):
        xz = jnp.dot(combined, inw[...], preferred_element_type=jnp.float32)
        xb = xz[:, :DI]
        z = xz[:, DI:]
        if first:
            c1 = jnp.zeros((BB, DI), jnp.float32)
            c2 = jnp.zeros((BB, DI), jnp.float32)
            c3 = jnp.zeros((BB, DI), jnp.float32)
        else:
            c1 = conv_ref[1]
            c2 = conv_ref[2]
            c3 = conv_ref[3]
        conv_ref[0] = c1
        conv_ref[1] = c2
        conv_ref[2] = c3
        conv_ref[3] = xb
        cw = convw[...]
        xc = (c1 * cw[0:1] + c2 * cw[1:2] + c3 * cw[2:3] + xb * cw[3:4]
              + convb[...])
        xc = xc * jax.nn.sigmoid(xc)
        xdb = jnp.dot(xc, xw[...], preferred_element_type=jnp.float32)
        dt = jax.nn.softplus(xdb[:, :DI] + dtb[...])
        Bm = xdb[:, DI:DI + ds]
        Cm = xdb[:, DI + ds:]
        dA = jnp.exp(dt[:, :, None] * A[...][None])
        w3 = (dt * xc)[:, :, None]
        if first:
            ssm = w3 * Bm[:, None, :]
        else:
            ssm = ssm_ref[...] * dA + w3 * Bm[:, None, :]
        ssm_ref[...] = ssm
        y = jnp.sum(ssm * Cm[:, None, :], axis=-1) + D[...] * xc
        y = y * (z * jax.nn.sigmoid(z))
        return jnp.dot(y, outw[...], preferred_element_type=jnp.float32)

    def step(t, carry, first=False):
        ws, fat = carry
        x_t = x_ref[t]
        combined = jnp.concatenate([x_t, ws], axis=-1)
        ov = mamba_step(combined, inw_v, convw_v, convb_v, xw_v, dtb_v,
                        A_v, D_v, outw_v, conv_v, ssm_v, DS[0], first)
        os_ = mamba_step(combined, inw_s, convw_s, convb_s, xw_s, dtb_s,
                         A_s, D_s, outw_s, conv_s, ssm_s, DS[1], first)
        oe = mamba_step(combined, inw_e, convw_e, convb_e, xw_e, dtb_e,
                        A_e, D_e, outw_e, conv_e, ssm_e, DS[2], first)
        gw = gatew_ref[...]
        votes = []
        scores = []
        for k, o in enumerate((ov, os_, oe)):
            weak = o * (1.0 - fat[:, k:k + 1])
            v = (jnp.dot(weak, projw_ref[k], preferred_element_type=jnp.float32)
                 + projb_ref[k])
            votes.append(v)
            scores.append(jnp.sum(v * gw, axis=-1, keepdims=True)
                          + gateb_ref[0, 0])
        sc = jnp.concatenate(scores, axis=-1) * (1.0 / TEMP)
        sc = sc - jnp.max(sc, axis=-1, keepdims=True)
        e = jnp.exp(sc)
        w = e / jnp.sum(e, axis=-1, keepdims=True)
        gc = (votes[0] * w[:, 0:1] + votes[1] * w[:, 1:2]
              + votes[2] * w[:, 2:3])
        fat = jnp.clip((fat + w * 0.15) * 0.85, 0.0, 0.6)
        mu = jnp.mean(gc, axis=-1, keepdims=True)
        d = gc - mu
        var = jnp.mean(d * d, axis=-1, keepdims=True)
        ws_new = d * jax.lax.rsqrt(var + LN_EPS) * lng_ref[...] + lnb_ref[...]
        out = (jnp.sum(ws_new * headw_ref[...], axis=-1) + headb_ref[0, 0])
        outs_ref[0, t] = out
        gates_ref[t] = w
        return ws_new, fat

    ws0 = jnp.zeros((BB, WS), jnp.float32)
    fat0 = jnp.zeros((BB, 3), jnp.float32)
    carry = step(0, (ws0, fat0), first=True)
    jax.lax.fori_loop(1, S, step, carry)


def kernel(x, visual_params, semantic_params, episodic_params, ws_params):
    f32 = jnp.float32

    def expert_inputs(p, ds):
        wdt = p['dt_proj_w'] @ p['x_proj_w'][:DT_RANK]
        wbc = p['x_proj_w'][DT_RANK:]
        xw = jnp.concatenate([wdt, wbc], axis=0).T
        return (
            p['in_proj_w'].T.astype(f32),
            p['conv_w'].T.astype(f32),
            p['conv_b'].reshape(1, DI).astype(f32),
            xw.astype(f32),
            p['dt_proj_b'].reshape(1, DI).astype(f32),
            (-jnp.exp(p['A_log'])).astype(f32),
            p['D'].reshape(1, DI).astype(f32),
            p['out_proj_w'].T.astype(f32),
        )

    wp = ws_params
    args = (
        jnp.swapaxes(x, 0, 1).astype(f32),
        *expert_inputs(visual_params, DS[0]),
        *expert_inputs(semantic_params, DS[1]),
        *expert_inputs(episodic_params, DS[2]),
        jnp.swapaxes(wp['proj_w'], 1, 2).astype(f32),
        wp['proj_b'].reshape(3, 1, WS).astype(f32),
        wp['gate_w'].reshape(1, WS).astype(f32),
        wp['gate_b'].reshape(1, 1).astype(f32),
        wp['ln_g'].reshape(1, WS).astype(f32),
        wp['ln_b'].reshape(1, WS).astype(f32),
        wp['head_w'].reshape(1, WS).astype(f32),
        wp['head_b'].reshape(1, 1).astype(f32),
    )

    def rep(shape):
        n = len(shape)
        return pl.BlockSpec(shape, lambda i, _n=n: (0,) * _n)

    in_specs = [pl.BlockSpec((S, BB, INPUT_DIM), lambda i: (0, i, 0))]
    for a in args[1:]:
        in_specs.append(rep(a.shape))

    outs, gates = pl.pallas_call(
        _thalamus_kernel,
        grid=(NCORE,),
        in_specs=in_specs,
        out_specs=[
            pl.BlockSpec((1, S, BB), lambda i: (i, 0, 0)),
            pl.BlockSpec((S, BB, 3), lambda i: (0, i, 0)),
        ],
        out_shape=[
            jax.ShapeDtypeStruct((NCORE, S, BB), f32),
            jax.ShapeDtypeStruct((S, B, 3), f32),
        ],
        scratch_shapes=[
            pltpu.VMEM((DCONV, BB, DI), f32),
            pltpu.VMEM((DCONV, BB, DI), f32),
            pltpu.VMEM((DCONV, BB, DI), f32),
            pltpu.VMEM((BB, DI, DS[0]), f32),
            pltpu.VMEM((BB, DI, DS[1]), f32),
            pltpu.VMEM((BB, DI, DS[2]), f32),
        ],
        compiler_params=pltpu.CompilerParams(
            dimension_semantics=("parallel",),
            vmem_limit_bytes=100 * 1024 * 1024,
        ),
        name="thalamus_fused",
    )(*args)

    outs = jnp.swapaxes(outs, 1, 2).reshape(B, S)
    return outs[:, :, None], gates[:, :, :, None]


# speedup vs baseline: 6.3175x; 3.3747x over previous
import jax
import jax.numpy as jnp
from jax.experimental import pallas as pl
from jax.experimental.pallas import tpu as pltpu

B, S, INPUT_DIM, WS = 64, 256, 2, 64
DM = INPUT_DIM + WS
DI = 2 * DM
DCONV = 4
DT_RANK = 5
TEMP = 0.05
LN_EPS = 1e-5
DS = (16, 64, 128)
NCORE = 2
BB = B // NCORE


def _thalamus_kernel(
    x_ref,
    inw_v, convw_v, convb_v, xw_v, dtb_v, A_v, D_v, outw_v,
    inw_s, convw_s, convb_s, xw_s, dtb_s, A_s, D_s, outw_s,
    inw_e, convw_e, convb_e, xw_e, dtb_e, A_e, D_e, outw_e,
    projw_ref, projb_ref, gatew_ref, gateb_ref,
    lng_ref, lnb_ref, headw_ref, headb_ref,
    outs_ref,
    gates_ref,
    conv_v, conv_s, conv_e,
    ssm_v, ssm_s, ssm_e,
):
    def mamba_step(combined, inw, convw, convb, xw, dtb, A, D, outw,
                   conv_ref, ssm_ref, ds, first):
        xz = jnp.dot(combined, inw[...], preferred_element_type=jnp.float32)
        xb = xz[:, :DI]
        z = xz[:, DI:]
        if first:
            c1 = jnp.zeros((BB, DI), jnp.float32)
            c2 = jnp.zeros((BB, DI), jnp.float32)
            c3 = jnp.zeros((BB, DI), jnp.float32)
        else:
            c1 = conv_ref[1]
            c2 = conv_ref[2]
            c3 = conv_ref[3]
        conv_ref[0] = c1
        conv_ref[1] = c2
        conv_ref[2] = c3
        conv_ref[3] = xb
        cw = convw[...]
        xc = (c1 * cw[0:1] + c2 * cw[1:2] + c3 * cw[2:3] + xb * cw[3:4]
              + convb[...])
        xc = xc * jax.nn.sigmoid(xc)
        xdb = jnp.dot(xc, xw[...], preferred_element_type=jnp.float32)
        dt = jax.nn.softplus(xdb[:, :DI] + dtb[...])
        Bm = xdb[:, 256:256 + ds]
        Cm = xdb[:, 384:384 + ds]
        dA = jnp.exp(dt[:, None, :] * A[...][None])
        w3 = (dt * xc)[:, None, :]
        Bm3 = jax.lax.broadcast_in_dim(Bm, (BB, ds, DI), (0, 1))
        Cm3 = jax.lax.broadcast_in_dim(Cm, (BB, ds, DI), (0, 1))
        if first:
            ssm = w3 * Bm3
        else:
            ssm = ssm_ref[...] * dA + w3 * Bm3
        ssm_ref[...] = ssm
        y = jnp.sum(ssm * Cm3, axis=1) + D[...] * xc
        y = y * (z * jax.nn.sigmoid(z))
        return jnp.dot(y, outw[...], preferred_element_type=jnp.float32)

    def step(t, carry, first=False):
        ws, fat = carry
        x_t = x_ref[t]
        combined = jnp.concatenate([x_t, ws], axis=-1)
        ov = mamba_step(combined, inw_v, convw_v, convb_v, xw_v, dtb_v,
                        A_v, D_v, outw_v, conv_v, ssm_v, DS[0], first)
        os_ = mamba_step(combined, inw_s, convw_s, convb_s, xw_s, dtb_s,
                         A_s, D_s, outw_s, conv_s, ssm_s, DS[1], first)
        oe = mamba_step(combined, inw_e, convw_e, convb_e, xw_e, dtb_e,
                        A_e, D_e, outw_e, conv_e, ssm_e, DS[2], first)
        gw = gatew_ref[...]
        votes = []
        scores = []
        for k, o in enumerate((ov, os_, oe)):
            weak = o * (1.0 - fat[:, k:k + 1])
            v = (jnp.dot(weak, projw_ref[k], preferred_element_type=jnp.float32)
                 + projb_ref[k])
            votes.append(v)
            scores.append(jnp.sum(v * gw, axis=-1, keepdims=True)
                          + gateb_ref[0, 0])
        sc = jnp.concatenate(scores, axis=-1) * (1.0 / TEMP)
        sc = sc - jnp.max(sc, axis=-1, keepdims=True)
        e = jnp.exp(sc)
        w = e / jnp.sum(e, axis=-1, keepdims=True)
        gc = (votes[0] * w[:, 0:1] + votes[1] * w[:, 1:2]
              + votes[2] * w[:, 2:3])
        fat = jnp.clip((fat + w * 0.15) * 0.85, 0.0, 0.6)
        mu = jnp.mean(gc, axis=-1, keepdims=True)
        d = gc - mu
        var = jnp.mean(d * d, axis=-1, keepdims=True)
        ws_new = d * jax.lax.rsqrt(var + LN_EPS) * lng_ref[...] + lnb_ref[...]
        out = (jnp.sum(ws_new * headw_ref[...], axis=-1) + headb_ref[0, 0])
        outs_ref[0, t] = out
        gates_ref[t] = w
        return ws_new, fat

    ws0 = jnp.zeros((BB, WS), jnp.float32)
    fat0 = jnp.zeros((BB, 3), jnp.float32)
    carry = step(0, (ws0, fat0), first=True)
    jax.lax.fori_loop(1, S, step, carry)


def kernel(x, visual_params, semantic_params, episodic_params, ws_params):
    f32 = jnp.float32

    def expert_inputs(p, ds):
        wdt = p['dt_proj_w'] @ p['x_proj_w'][:DT_RANK]
        wb = p['x_proj_w'][DT_RANK:DT_RANK + ds]
        wc = p['x_proj_w'][DT_RANK + ds:]
        z1 = jnp.zeros((256 - DI, DI))
        z2 = jnp.zeros((128 - ds, DI)) if ds < 128 else jnp.zeros((0, DI))
        xw = jnp.concatenate([wdt, z1, wb, z2, wc, z2], axis=0).T
        return (
            p['in_proj_w'].T.astype(f32),
            p['conv_w'].T.astype(f32),
            p['conv_b'].reshape(1, DI).astype(f32),
            xw.astype(f32),
            p['dt_proj_b'].reshape(1, DI).astype(f32),
            (-jnp.exp(p['A_log'])).T.astype(f32),
            p['D'].reshape(1, DI).astype(f32),
            p['out_proj_w'].T.astype(f32),
        )

    wp = ws_params
    args = (
        jnp.swapaxes(x, 0, 1).astype(f32),
        *expert_inputs(visual_params, DS[0]),
        *expert_inputs(semantic_params, DS[1]),
        *expert_inputs(episodic_params, DS[2]),
        jnp.swapaxes(wp['proj_w'], 1, 2).astype(f32),
        wp['proj_b'].reshape(3, 1, WS).astype(f32),
        wp['gate_w'].reshape(1, WS).astype(f32),
        wp['gate_b'].reshape(1, 1).astype(f32),
        wp['ln_g'].reshape(1, WS).astype(f32),
        wp['ln_b'].reshape(1, WS).astype(f32),
        wp['head_w'].reshape(1, WS).astype(f32),
        wp['head_b'].reshape(1, 1).astype(f32),
    )

    def rep(shape):
        n = len(shape)
        return pl.BlockSpec(shape, lambda i, _n=n: (0,) * _n)

    in_specs = [pl.BlockSpec((S, BB, INPUT_DIM), lambda i: (0, i, 0))]
    for a in args[1:]:
        in_specs.append(rep(a.shape))

    outs, gates = pl.pallas_call(
        _thalamus_kernel,
        grid=(NCORE,),
        in_specs=in_specs,
        out_specs=[
            pl.BlockSpec((1, S, BB), lambda i: (i, 0, 0)),
            pl.BlockSpec((S, BB, 3), lambda i: (0, i, 0)),
        ],
        out_shape=[
            jax.ShapeDtypeStruct((NCORE, S, BB), f32),
            jax.ShapeDtypeStruct((S, B, 3), f32),
        ],
        scratch_shapes=[
            pltpu.VMEM((DCONV, BB, DI), f32),
            pltpu.VMEM((DCONV, BB, DI), f32),
            pltpu.VMEM((DCONV, BB, DI), f32),
            pltpu.VMEM((BB, DS[0], DI), f32),
            pltpu.VMEM((BB, DS[1], DI), f32),
            pltpu.VMEM((BB, DS[2], DI), f32),
        ],
        compiler_params=pltpu.CompilerParams(
            dimension_semantics=("parallel",),
            vmem_limit_bytes=100 * 1024 * 1024,
        ),
        name="thalamus_fused",
    )(*args)

    outs = jnp.swapaxes(outs, 1, 2).reshape(B, S)
    return outs[:, :, None], gates[:, :, :, None]
